# baseline (speedup 1.0000x reference)
import functools

import jax
import jax.numpy as jnp
from jax import lax
from jax.experimental import pallas as pl
from jax.experimental.pallas import tpu as pltpu
from jax.experimental.pallas import tpu_sc as plsc

N = 10000
E = 320000
D_IN = 128
D_EDGE = 16
H = 128
L = 3

NUM_CORES = 2
NUM_SUBCORES = 16
NUM_WORKERS = NUM_CORES * NUM_SUBCORES
CH = 80
EP = 322560
EPW = EP // NUM_WORKERS
NCH = EPW // CH
NBUF = 2
GRP = 14
GRP_OUT = GRP // NBUF
NGRP = NCH // GRP
NP = 10112
RPT = NP // NUM_SUBCORES
DUMP = N
ZB = 8
LANES = 16


def _sc_agg_body(h_hbm, e_hbm, src_hbm, dst_hbm, out_hbm,
                 dsti, srci, b0, b1, zbuf, aggs, *sems):
    bufs = (b0, b1)
    esem = sems[0:NBUF]
    gsem = sems[NBUF:2 * NBUF]
    ssem = sems[2 * NBUF:3 * NBUF]
    c = lax.axis_index("c")
    s = lax.axis_index("s")
    wid = c * NUM_SUBCORES + s

    zv = jnp.zeros((LANES,), jnp.float32)
    for r in range(ZB):
        for q in range(H // LANES):
            zbuf[r, pl.ds(q * LANES, LANES)] = zv

    def _zcopy(k, _):
        pltpu.sync_copy(zbuf, aggs.at[pl.ds(s * RPT + k * ZB, ZB)])
        return 0

    lax.fori_loop(0, RPT // ZB, _zcopy, 0)
    plsc.subcore_barrier()

    def _outer(k, _):
        @pl.when(k > 0)
        def _():
            for b in range(NBUF):
                pltpu.make_async_copy(
                    bufs[b], aggs.at[srci.at[b]], ssem[b]).wait()

        @pl.when(lax.rem(k, GRP_OUT) == 0)
        def _():
            g = lax.div(k, GRP_OUT)
            pltpu.sync_copy(dst_hbm.at[wid, g], dsti)
            pltpu.sync_copy(src_hbm.at[wid, g], srci)

        for b in range(NBUF):
            base = wid * EPW + (k * NBUF + b) * CH
            pltpu.async_copy(e_hbm.at[pl.ds(base, CH)], bufs[b], esem[b])

        for b in range(NBUF):
            jj = lax.rem(k, GRP_OUT) * NBUF + b
            base = wid * EPW + (k * NBUF + b) * CH
            pltpu.make_async_copy(
                e_hbm.at[pl.ds(base, CH)], bufs[b], esem[b]).wait()
            pltpu.async_copy(h_hbm.at[dsti.at[jj]], bufs[b], gsem[b],
                             add=True)

        for b in range(NBUF):
            jj = lax.rem(k, GRP_OUT) * NBUF + b
            pltpu.make_async_copy(
                h_hbm.at[dsti.at[jj]], bufs[b], gsem[b]).wait()

            def _relu_row(r, _, b=b):
                for q in range(H // LANES):
                    sl = pl.ds(q * LANES, LANES)
                    bufs[b][r, sl] = jnp.maximum(bufs[b][r, sl], 0.0)
                return 0

            lax.fori_loop(0, CH, _relu_row, 0)
            pltpu.async_copy(bufs[b], aggs.at[srci.at[jj]], ssem[b], add=True)
        return 0

    lax.fori_loop(0, NCH // NBUF, _outer, 0)
    for b in range(NBUF):
        pltpu.make_async_copy(bufs[b], aggs.at[srci.at[b]], ssem[b]).wait()
    plsc.subcore_barrier()

    pltpu.sync_copy(aggs.at[pl.ds(s * RPT, RPT)],
                    out_hbm.at[c, pl.ds(s * RPT, RPT)])


@functools.cache
def _sc_agg():
    return pl.kernel(
        _sc_agg_body,
        out_type=jax.ShapeDtypeStruct((NUM_CORES, NP, H), jnp.float32),
        mesh=plsc.VectorSubcoreMesh(
            core_axis_name="c", subcore_axis_name="s",
            num_cores=NUM_CORES, num_subcores=NUM_SUBCORES,
        ),
        scratch_types=[
            pltpu.VMEM((GRP, CH), jnp.int32),
            pltpu.VMEM((GRP, CH), jnp.int32),
        ] + [pltpu.VMEM((CH, H), jnp.float32) for _ in range(NBUF)] + [
            pltpu.VMEM((ZB, H), jnp.float32),
            pltpu.VMEM_SHARED((NP, H), jnp.float32),
        ] + [pltpu.SemaphoreType.DMA for _ in range(3 * NBUF)],
    )


def _proj_body(x_ref, w_ref, b_ref, o_ref):
    o_ref[...] = (
        jnp.dot(x_ref[...], w_ref[...], preferred_element_type=jnp.float32)
        + b_ref[...]
    )


def _proj(x, w, b, block_rows):
    rows, d_in = x.shape
    grid = rows // block_rows
    return pl.pallas_call(
        _proj_body,
        grid=(grid,),
        in_specs=[
            pl.BlockSpec((block_rows, d_in), lambda i: (i, 0)),
            pl.BlockSpec((d_in, H), lambda i: (0, 0)),
            pl.BlockSpec((1, H), lambda i: (0, 0)),
        ],
        out_specs=pl.BlockSpec((block_rows, H), lambda i: (i, 0)),
        out_shape=jax.ShapeDtypeStruct((rows, H), jnp.float32),
    )(x, w, b.reshape(1, H))


def _mlp_body(h_ref, a_ref, w1_ref, b1_ref, w2_ref, b2_ref, g_ref, be_ref,
              o_ref):
    h = h_ref[...]
    new = h + a_ref[0] + a_ref[1]
    hid = jax.nn.gelu(
        jnp.dot(new, w1_ref[...], preferred_element_type=jnp.float32)
        + b1_ref[...]
    )
    new = (
        jnp.dot(hid, w2_ref[...], preferred_element_type=jnp.float32)
        + b2_ref[...]
    )
    x = new + h
    mu = jnp.mean(x, axis=-1, keepdims=True)
    var = jnp.mean((x - mu) ** 2, axis=-1, keepdims=True)
    o_ref[...] = (x - mu) / jnp.sqrt(var + 1e-5) * g_ref[...] + be_ref[...]


def _mlp(h, agg2, w1, b1, w2, b2, g, be, block_rows=1000):
    grid = N // block_rows
    return pl.pallas_call(
        _mlp_body,
        grid=(grid,),
        in_specs=[
            pl.BlockSpec((block_rows, H), lambda i: (i, 0)),
            pl.BlockSpec((NUM_CORES, block_rows, H), lambda i: (0, i, 0)),
            pl.BlockSpec((H, H // 2), lambda i: (0, 0)),
            pl.BlockSpec((1, H // 2), lambda i: (0, 0)),
            pl.BlockSpec((H // 2, H), lambda i: (0, 0)),
            pl.BlockSpec((1, H), lambda i: (0, 0)),
            pl.BlockSpec((1, H), lambda i: (0, 0)),
            pl.BlockSpec((1, H), lambda i: (0, 0)),
        ],
        out_specs=pl.BlockSpec((block_rows, H), lambda i: (i, 0)),
        out_shape=jax.ShapeDtypeStruct((N, H), jnp.float32),
    )(h, agg2, w1, b1.reshape(1, H // 2), w2, b2.reshape(1, H),
      g.reshape(1, H), be.reshape(1, H))


def kernel(node_feats, edge_feats, edge_index, W_node, b_node, W_edge, b_edge,
           W1, b1, W2, b2, gamma, beta):
    pad = EP - E
    src = jnp.concatenate(
        [edge_index[0].astype(jnp.int32),
         jnp.full((pad,), DUMP, jnp.int32)]).reshape(NUM_WORKERS, NGRP, GRP, CH)
    dst = jnp.concatenate(
        [edge_index[1].astype(jnp.int32),
         jnp.zeros((pad,), jnp.int32)]).reshape(NUM_WORKERS, NGRP, GRP, CH)
    ef = jnp.concatenate(
        [edge_feats, jnp.zeros((pad, D_EDGE), jnp.float32)])

    h = _proj(node_feats, W_node, b_node, block_rows=1000)
    e = _proj(ef, W_edge, b_edge, block_rows=1920)

    def layer(h, wts):
        w1, bb1, w2, bb2, g, be = wts
        agg2 = _sc_agg()(h, e, src, dst)
        h = _mlp(h, agg2, w1, bb1, w2, bb2, g, be)
        return h, None

    h, _ = lax.scan(layer, h, (W1, b1, W2, b2, gamma, beta))
    return h


# speedup vs baseline: 1.1129x; 1.1129x over previous
import functools

import jax
import jax.numpy as jnp
from jax import lax
from jax.experimental import pallas as pl
from jax.experimental.pallas import tpu as pltpu
from jax.experimental.pallas import tpu_sc as plsc

N = 10000
E = 320000
D_IN = 128
D_EDGE = 16
H = 128
L = 3

NUM_CORES = 2
NUM_SUBCORES = 16
NUM_WORKERS = NUM_CORES * NUM_SUBCORES
CH = 56
NCH = 180
EPW = NCH * CH
EP = EPW * NUM_WORKERS
NBUF = 3
NOUT = NCH // NBUF
GRP = 15
GRP_OUT = GRP // NBUF
NGRP = NCH // GRP
NP = 10112
RPT = NP // NUM_SUBCORES
DUMP = N
LANES = 16


def _sc_agg_body(h_hbm, e_hbm, src_hbm, dst_hbm, out_hbm,
                 dsti, srci, b0, b1, b2, aggs, *sems):
    bufs = (b0, b1, b2)
    esem = sems[0:NBUF]
    gsem = sems[NBUF:2 * NBUF]
    ssem = sems[2 * NBUF:3 * NBUF]
    c = lax.axis_index("c")
    s = lax.axis_index("s")
    wid = c * NUM_SUBCORES + s

    zv = jnp.zeros((LANES,), jnp.float32)
    for r in range(CH):
        for q in range(H // LANES):
            b0[r, pl.ds(q * LANES, LANES)] = zv
    for i in range(RPT // CH):
        pltpu.sync_copy(b0, aggs.at[pl.ds(s * RPT + i * CH, CH)])
    pltpu.sync_copy(b0.at[pl.ds(0, RPT % CH)],
                    aggs.at[pl.ds(s * RPT + (RPT // CH) * CH, RPT % CH)])
    plsc.subcore_barrier()

    for b in range(NBUF):
        base = wid * EPW + b * CH
        pltpu.async_copy(e_hbm.at[pl.ds(base, CH)], bufs[b], esem[b])

    def _outer(k, _):
        @pl.when(lax.rem(k, GRP_OUT) == 0)
        def _():
            g = lax.div(k, GRP_OUT)
            pltpu.sync_copy(dst_hbm.at[wid, g], dsti)
            pltpu.sync_copy(src_hbm.at[wid, g], srci)

        for b in range(NBUF):
            jj = lax.rem(k, GRP_OUT) * NBUF + b
            base = wid * EPW + (k * NBUF + b) * CH
            pltpu.make_async_copy(
                e_hbm.at[pl.ds(base, CH)], bufs[b], esem[b]).wait()
            pltpu.async_copy(h_hbm.at[dsti.at[jj]], bufs[b], gsem[b],
                             add=True)

        for b in range(NBUF):
            jj = lax.rem(k, GRP_OUT) * NBUF + b
            pltpu.make_async_copy(
                h_hbm.at[dsti.at[jj]], bufs[b], gsem[b]).wait()

            def _relu_row(r, _, b=b):
                for q in range(H // LANES):
                    sl = pl.ds(q * LANES, LANES)
                    bufs[b][r, sl] = jnp.maximum(bufs[b][r, sl], 0.0)
                return 0

            lax.fori_loop(0, CH, _relu_row, 0)
            pltpu.async_copy(bufs[b], aggs.at[srci.at[jj]], ssem[b], add=True)

        for b in range(NBUF):
            jj = lax.rem(k, GRP_OUT) * NBUF + b
            pltpu.make_async_copy(
                bufs[b], aggs.at[srci.at[jj]], ssem[b]).wait()

            @pl.when(k + 1 < NOUT)
            def _(b=b):
                base = wid * EPW + ((k + 1) * NBUF + b) * CH
                pltpu.async_copy(e_hbm.at[pl.ds(base, CH)], bufs[b], esem[b])
        return 0

    lax.fori_loop(0, NOUT, _outer, 0)
    plsc.subcore_barrier()

    pltpu.sync_copy(aggs.at[pl.ds(s * RPT, RPT)],
                    out_hbm.at[c, pl.ds(s * RPT, RPT)])


@functools.cache
def _sc_agg():
    return pl.kernel(
        _sc_agg_body,
        out_type=jax.ShapeDtypeStruct((NUM_CORES, NP, H), jnp.float32),
        mesh=plsc.VectorSubcoreMesh(
            core_axis_name="c", subcore_axis_name="s",
            num_cores=NUM_CORES, num_subcores=NUM_SUBCORES,
        ),
        scratch_types=[
            pltpu.VMEM((GRP, CH), jnp.int32),
            pltpu.VMEM((GRP, CH), jnp.int32),
        ] + [pltpu.VMEM((CH, H), jnp.float32) for _ in range(NBUF)] + [
            pltpu.VMEM_SHARED((NP, H), jnp.float32),
        ] + [pltpu.SemaphoreType.DMA for _ in range(3 * NBUF)],
    )


def _proj_body(x_ref, w_ref, b_ref, o_ref):
    o_ref[...] = (
        jnp.dot(x_ref[...], w_ref[...], preferred_element_type=jnp.float32)
        + b_ref[...]
    )


def _proj(x, w, b, block_rows):
    rows, d_in = x.shape
    grid = rows // block_rows
    return pl.pallas_call(
        _proj_body,
        grid=(grid,),
        in_specs=[
            pl.BlockSpec((block_rows, d_in), lambda i: (i, 0)),
            pl.BlockSpec((d_in, H), lambda i: (0, 0)),
            pl.BlockSpec((1, H), lambda i: (0, 0)),
        ],
        out_specs=pl.BlockSpec((block_rows, H), lambda i: (i, 0)),
        out_shape=jax.ShapeDtypeStruct((rows, H), jnp.float32),
    )(x, w, b.reshape(1, H))


def _mlp_body(h_ref, a_ref, w1_ref, b1_ref, w2_ref, b2_ref, g_ref, be_ref,
              o_ref):
    h = h_ref[...]
    new = h + a_ref[0] + a_ref[1]
    hid = jax.nn.gelu(
        jnp.dot(new, w1_ref[...], preferred_element_type=jnp.float32)
        + b1_ref[...]
    )
    new = (
        jnp.dot(hid, w2_ref[...], preferred_element_type=jnp.float32)
        + b2_ref[...]
    )
    x = new + h
    mu = jnp.mean(x, axis=-1, keepdims=True)
    var = jnp.mean((x - mu) ** 2, axis=-1, keepdims=True)
    o_ref[...] = (x - mu) / jnp.sqrt(var + 1e-5) * g_ref[...] + be_ref[...]


def _mlp(h, agg2, w1, b1, w2, b2, g, be, block_rows=1000):
    grid = N // block_rows
    return pl.pallas_call(
        _mlp_body,
        grid=(grid,),
        in_specs=[
            pl.BlockSpec((block_rows, H), lambda i: (i, 0)),
            pl.BlockSpec((NUM_CORES, block_rows, H), lambda i: (0, i, 0)),
            pl.BlockSpec((H, H // 2), lambda i: (0, 0)),
            pl.BlockSpec((1, H // 2), lambda i: (0, 0)),
            pl.BlockSpec((H // 2, H), lambda i: (0, 0)),
            pl.BlockSpec((1, H), lambda i: (0, 0)),
            pl.BlockSpec((1, H), lambda i: (0, 0)),
            pl.BlockSpec((1, H), lambda i: (0, 0)),
        ],
        out_specs=pl.BlockSpec((block_rows, H), lambda i: (i, 0)),
        out_shape=jax.ShapeDtypeStruct((N, H), jnp.float32),
    )(h, agg2, w1, b1.reshape(1, H // 2), w2, b2.reshape(1, H),
      g.reshape(1, H), be.reshape(1, H))


def kernel(node_feats, edge_feats, edge_index, W_node, b_node, W_edge, b_edge,
           W1, b1, W2, b2, gamma, beta):
    pad = EP - E
    src = jnp.concatenate(
        [edge_index[0].astype(jnp.int32),
         jnp.full((pad,), DUMP, jnp.int32)]).reshape(NUM_WORKERS, NGRP, GRP, CH)
    dst = jnp.concatenate(
        [edge_index[1].astype(jnp.int32),
         jnp.zeros((pad,), jnp.int32)]).reshape(NUM_WORKERS, NGRP, GRP, CH)
    ef = jnp.concatenate(
        [edge_feats, jnp.zeros((pad, D_EDGE), jnp.float32)])

    h = _proj(node_feats, W_node, b_node, block_rows=1000)
    e = _proj(ef, W_edge, b_edge, block_rows=1920)

    def layer(h, wts):
        w1, bb1, w2, bb2, g, be = wts
        agg2 = _sc_agg()(h, e, src, dst)
        h = _mlp(h, agg2, w1, bb1, w2, bb2, g, be)
        return h, None

    h, _ = lax.scan(layer, h, (W1, b1, W2, b2, gamma, beta))
    return h


# speedup vs baseline: 1.1360x; 1.0207x over previous
import functools

import jax
import jax.numpy as jnp
from jax import lax
from jax.experimental import pallas as pl
from jax.experimental.pallas import tpu as pltpu
from jax.experimental.pallas import tpu_sc as plsc

N = 10000
E = 320000
D_IN = 128
D_EDGE = 16
H = 128
L = 3

NUM_CORES = 2
NUM_SUBCORES = 16
NUM_WORKERS = NUM_CORES * NUM_SUBCORES
CH = 56
NCH = 180
EPW = NCH * CH
EP = EPW * NUM_WORKERS
NBUF = 3
NOUT = NCH // NBUF
GRP = 15
GRP_OUT = GRP // NBUF
NGRP = NCH // GRP
NP = 10112
RPT = NP // NUM_SUBCORES
DUMP = N
LANES = 16


def _sc_agg_body(h_hbm, e_hbm, src_hbm, dst_hbm, out_hbm,
                 dsti, srci, b0, b1, b2, aggs, *sems):
    bufs = (b0, b1, b2)
    esem = sems[0:NBUF]
    gsem = sems[NBUF:2 * NBUF]
    ssem = sems[2 * NBUF:3 * NBUF]
    c = lax.axis_index("c")
    s = lax.axis_index("s")
    wid = c * NUM_SUBCORES + s

    zv = jnp.zeros((LANES,), jnp.float32)
    for r in range(CH):
        for q in range(H // LANES):
            b0[r, pl.ds(q * LANES, LANES)] = zv
    for i in range(RPT // CH):
        pltpu.sync_copy(b0, aggs.at[pl.ds(s * RPT + i * CH, CH)])
    pltpu.sync_copy(b0.at[pl.ds(0, RPT % CH)],
                    aggs.at[pl.ds(s * RPT + (RPT // CH) * CH, RPT % CH)])
    plsc.subcore_barrier()

    for b in range(NBUF):
        base = wid * EPW + b * CH
        pltpu.async_copy(e_hbm.at[pl.ds(base, CH)], bufs[b], esem[b])

    def _outer(k, _):
        @pl.when(lax.rem(k, GRP_OUT) == 0)
        def _():
            g = lax.div(k, GRP_OUT)
            pltpu.sync_copy(dst_hbm.at[wid, g], dsti)
            pltpu.sync_copy(src_hbm.at[wid, g], srci)

        for b in range(NBUF):
            jj = lax.rem(k, GRP_OUT) * NBUF + b
            base = wid * EPW + (k * NBUF + b) * CH
            pltpu.make_async_copy(
                e_hbm.at[pl.ds(base, CH)], bufs[b], esem[b]).wait()
            pltpu.async_copy(h_hbm.at[dsti.at[jj]], bufs[b], gsem[b],
                             add=True)

        for b in range(NBUF):
            jj = lax.rem(k, GRP_OUT) * NBUF + b
            pltpu.make_async_copy(
                h_hbm.at[dsti.at[jj]], bufs[b], gsem[b]).wait()

            def _relu_row(r, _, b=b):
                for q in range(H // LANES):
                    sl = pl.ds(q * LANES, LANES)
                    bufs[b][r, sl] = jnp.maximum(bufs[b][r, sl], 0.0)
                return 0

            lax.fori_loop(0, CH, _relu_row, 0)
            pltpu.async_copy(bufs[b], aggs.at[srci.at[jj]], ssem[b], add=True)

        for b in range(NBUF):
            jj = lax.rem(k, GRP_OUT) * NBUF + b
            pltpu.make_async_copy(
                bufs[b], aggs.at[srci.at[jj]], ssem[b]).wait()

            @pl.when(k + 1 < NOUT)
            def _(b=b):
                base = wid * EPW + ((k + 1) * NBUF + b) * CH
                pltpu.async_copy(e_hbm.at[pl.ds(base, CH)], bufs[b], esem[b])
        return 0

    lax.fori_loop(0, NOUT, _outer, 0)
    plsc.subcore_barrier()

    pltpu.sync_copy(aggs.at[pl.ds(s * RPT, RPT)],
                    out_hbm.at[c, pl.ds(s * RPT, RPT)])


@functools.cache
def _sc_agg():
    return pl.kernel(
        _sc_agg_body,
        out_type=jax.ShapeDtypeStruct((NUM_CORES, NP, H), jnp.float32),
        mesh=plsc.VectorSubcoreMesh(
            core_axis_name="c", subcore_axis_name="s",
            num_cores=NUM_CORES, num_subcores=NUM_SUBCORES,
        ),
        scratch_types=[
            pltpu.VMEM((GRP, CH), jnp.int32),
            pltpu.VMEM((GRP, CH), jnp.int32),
        ] + [pltpu.VMEM((CH, H), jnp.float32) for _ in range(NBUF)] + [
            pltpu.VMEM_SHARED((NP, H), jnp.float32),
        ] + [pltpu.SemaphoreType.DMA for _ in range(3 * NBUF)],
    )


def _proj_body(x_ref, w_ref, b_ref, o_ref):
    o_ref[...] = (
        jnp.dot(x_ref[...], w_ref[...], preferred_element_type=jnp.float32)
        + b_ref[...]
    )


def _proj(x, w, b, block_rows):
    rows, d_in = x.shape
    grid = rows // block_rows
    return pl.pallas_call(
        _proj_body,
        grid=(grid,),
        in_specs=[
            pl.BlockSpec((block_rows, d_in), lambda i: (i, 0)),
            pl.BlockSpec((d_in, H), lambda i: (0, 0)),
            pl.BlockSpec((1, H), lambda i: (0, 0)),
        ],
        out_specs=pl.BlockSpec((block_rows, H), lambda i: (i, 0)),
        out_shape=jax.ShapeDtypeStruct((rows, H), jnp.float32),
    )(x, w, b.reshape(1, H))


def _mlp_body(h_ref, a_ref, w1_ref, b1_ref, w2_ref, b2_ref, g_ref, be_ref,
              o_ref):
    h = h_ref[...]
    new = h + a_ref[0] + a_ref[1]
    hid = jax.nn.gelu(
        jnp.dot(new, w1_ref[...], preferred_element_type=jnp.float32)
        + b1_ref[...]
    )
    new = (
        jnp.dot(hid, w2_ref[...], preferred_element_type=jnp.float32)
        + b2_ref[...]
    )
    x = new + h
    mu = jnp.mean(x, axis=-1, keepdims=True)
    var = jnp.mean((x - mu) ** 2, axis=-1, keepdims=True)
    o_ref[...] = (x - mu) / jnp.sqrt(var + 1e-5) * g_ref[...] + be_ref[...]


def _mlp(h, agg2, w1, b1, w2, b2, g, be, block_rows=1000):
    grid = N // block_rows
    return pl.pallas_call(
        _mlp_body,
        grid=(grid,),
        in_specs=[
            pl.BlockSpec((block_rows, H), lambda i: (i, 0)),
            pl.BlockSpec((NUM_CORES, block_rows, H), lambda i: (0, i, 0)),
            pl.BlockSpec((H, H // 2), lambda i: (0, 0)),
            pl.BlockSpec((1, H // 2), lambda i: (0, 0)),
            pl.BlockSpec((H // 2, H), lambda i: (0, 0)),
            pl.BlockSpec((1, H), lambda i: (0, 0)),
            pl.BlockSpec((1, H), lambda i: (0, 0)),
            pl.BlockSpec((1, H), lambda i: (0, 0)),
        ],
        out_specs=pl.BlockSpec((block_rows, H), lambda i: (i, 0)),
        out_shape=jax.ShapeDtypeStruct((N, H), jnp.float32),
    )(h, agg2, w1, b1.reshape(1, H // 2), w2, b2.reshape(1, H),
      g.reshape(1, H), be.reshape(1, H))


def kernel(node_feats, edge_feats, edge_index, W_node, b_node, W_edge, b_edge,
           W1, b1, W2, b2, gamma, beta):
    pad = EP - E
    pad_src = DUMP + jnp.arange(pad, dtype=jnp.int32) % (NP - N)
    src = jnp.concatenate(
        [edge_index[0].astype(jnp.int32),
         pad_src]).reshape(NUM_WORKERS, NGRP, GRP, CH)
    dst = jnp.concatenate(
        [edge_index[1].astype(jnp.int32),
         jnp.zeros((pad,), jnp.int32)]).reshape(NUM_WORKERS, NGRP, GRP, CH)
    ef = jnp.concatenate(
        [edge_feats, jnp.zeros((pad, D_EDGE), jnp.float32)])

    h = _proj(node_feats, W_node, b_node, block_rows=1000)
    e = _proj(ef, W_edge, b_edge, block_rows=1920)

    def layer(h, wts):
        w1, bb1, w2, bb2, g, be = wts
        agg2 = _sc_agg()(h, e, src, dst)
        h = _mlp(h, agg2, w1, bb1, w2, bb2, g, be)
        return h, None

    h, _ = lax.scan(layer, h, (W1, b1, W2, b2, gamma, beta))
    return h


# speedup vs baseline: 1.2175x; 1.0718x over previous
import functools

import jax
import jax.numpy as jnp
from jax import lax
from jax.experimental import pallas as pl
from jax.experimental.pallas import tpu as pltpu
from jax.experimental.pallas import tpu_sc as plsc

N = 10000
E = 320000
D_IN = 128
D_EDGE = 16
H = 128
L = 3

NUM_CORES = 2
NUM_SUBCORES = 16
NUM_WORKERS = NUM_CORES * NUM_SUBCORES
CH = 56
NCH0 = 216
NCH1 = 144
TOTCH = NUM_SUBCORES * (NCH0 + NCH1)
EP = TOTCH * CH
NBUF = 3
GRP = 24
GRP_OUT = GRP // NBUF
NP = 10112
RPT = NP // NUM_SUBCORES
DUMP = N
LANES = 16


def _sc_agg_body(h_hbm, e_hbm, src_hbm, dst_hbm, out_hbm,
                 dsti, srci, b0, b1, b2, aggs, *sems):
    bufs = (b0, b1, b2)
    esem = sems[0:NBUF]
    gsem = sems[NBUF:2 * NBUF]
    ssem = sems[2 * NBUF:3 * NBUF]
    c = lax.axis_index("c")
    s = lax.axis_index("s")
    crow = jnp.where(c == 0, s * NCH0, NUM_SUBCORES * NCH0 + s * NCH1)
    nout = jnp.where(c == 0, NCH0 // NBUF, NCH1 // NBUF)

    zv = jnp.zeros((LANES,), jnp.float32)
    for r in range(CH):
        for q in range(H // LANES):
            b0[r, pl.ds(q * LANES, LANES)] = zv
    for i in range(RPT // CH):
        pltpu.sync_copy(b0, aggs.at[pl.ds(s * RPT + i * CH, CH)])
    pltpu.sync_copy(b0.at[pl.ds(0, RPT % CH)],
                    aggs.at[pl.ds(s * RPT + (RPT // CH) * CH, RPT % CH)])
    plsc.subcore_barrier()

    for b in range(NBUF):
        base = (crow + b) * CH
        pltpu.async_copy(e_hbm.at[pl.ds(base, CH)], bufs[b], esem[b])

    def _outer(k, _):
        @pl.when(lax.rem(k, GRP_OUT) == 0)
        def _():
            g = lax.div(k, GRP_OUT)
            pltpu.sync_copy(dst_hbm.at[pl.ds(crow + g * GRP, GRP)], dsti)
            pltpu.sync_copy(src_hbm.at[pl.ds(crow + g * GRP, GRP)], srci)

        for b in range(NBUF):
            jj = lax.rem(k, GRP_OUT) * NBUF + b
            base = (crow + k * NBUF + b) * CH
            pltpu.make_async_copy(
                e_hbm.at[pl.ds(base, CH)], bufs[b], esem[b]).wait()
            pltpu.async_copy(h_hbm.at[dsti.at[jj]], bufs[b], gsem[b],
                             add=True)

        for b in range(NBUF):
            jj = lax.rem(k, GRP_OUT) * NBUF + b
            pltpu.make_async_copy(
                h_hbm.at[dsti.at[jj]], bufs[b], gsem[b]).wait()

            def _relu_row(r, _, b=b):
                for q in range(H // LANES):
                    sl = pl.ds(q * LANES, LANES)
                    bufs[b][r, sl] = jnp.maximum(bufs[b][r, sl], 0.0)
                return 0

            lax.fori_loop(0, CH, _relu_row, 0)
            pltpu.async_copy(bufs[b], aggs.at[srci.at[jj]], ssem[b], add=True)

        for b in range(NBUF):
            jj = lax.rem(k, GRP_OUT) * NBUF + b
            pltpu.make_async_copy(
                bufs[b], aggs.at[srci.at[jj]], ssem[b]).wait()

            @pl.when(k + 1 < nout)
            def _(b=b):
                base = (crow + (k + 1) * NBUF + b) * CH
                pltpu.async_copy(e_hbm.at[pl.ds(base, CH)], bufs[b], esem[b])
        return 0

    lax.fori_loop(0, nout, _outer, 0)
    plsc.subcore_barrier()

    pltpu.sync_copy(aggs.at[pl.ds(s * RPT, RPT)],
                    out_hbm.at[c, pl.ds(s * RPT, RPT)])


@functools.cache
def _sc_agg():
    return pl.kernel(
        _sc_agg_body,
        out_type=jax.ShapeDtypeStruct((NUM_CORES, NP, H), jnp.float32),
        mesh=plsc.VectorSubcoreMesh(
            core_axis_name="c", subcore_axis_name="s",
            num_cores=NUM_CORES, num_subcores=NUM_SUBCORES,
        ),
        scratch_types=[
            pltpu.VMEM((GRP, CH), jnp.int32),
            pltpu.VMEM((GRP, CH), jnp.int32),
        ] + [pltpu.VMEM((CH, H), jnp.float32) for _ in range(NBUF)] + [
            pltpu.VMEM_SHARED((NP, H), jnp.float32),
        ] + [pltpu.SemaphoreType.DMA for _ in range(3 * NBUF)],
    )


def _proj_body(x_ref, w_ref, b_ref, o_ref):
    o_ref[...] = (
        jnp.dot(x_ref[...], w_ref[...], preferred_element_type=jnp.float32)
        + b_ref[...]
    )


def _proj(x, w, b, block_rows):
    rows, d_in = x.shape
    grid = rows // block_rows
    return pl.pallas_call(
        _proj_body,
        grid=(grid,),
        in_specs=[
            pl.BlockSpec((block_rows, d_in), lambda i: (i, 0)),
            pl.BlockSpec((d_in, H), lambda i: (0, 0)),
            pl.BlockSpec((1, H), lambda i: (0, 0)),
        ],
        out_specs=pl.BlockSpec((block_rows, H), lambda i: (i, 0)),
        out_shape=jax.ShapeDtypeStruct((rows, H), jnp.float32),
    )(x, w, b.reshape(1, H))


def _mlp_body(h_ref, a_ref, w1_ref, b1_ref, w2_ref, b2_ref, g_ref, be_ref,
              o_ref):
    h = h_ref[...]
    new = h + a_ref[0] + a_ref[1]
    hid = jax.nn.gelu(
        jnp.dot(new, w1_ref[...], preferred_element_type=jnp.float32)
        + b1_ref[...]
    )
    new = (
        jnp.dot(hid, w2_ref[...], preferred_element_type=jnp.float32)
        + b2_ref[...]
    )
    x = new + h
    mu = jnp.mean(x, axis=-1, keepdims=True)
    var = jnp.mean((x - mu) ** 2, axis=-1, keepdims=True)
    o_ref[...] = (x - mu) / jnp.sqrt(var + 1e-5) * g_ref[...] + be_ref[...]


def _mlp(h, agg2, w1, b1, w2, b2, g, be, block_rows=1000):
    grid = N // block_rows
    return pl.pallas_call(
        _mlp_body,
        grid=(grid,),
        in_specs=[
            pl.BlockSpec((block_rows, H), lambda i: (i, 0)),
            pl.BlockSpec((NUM_CORES, block_rows, H), lambda i: (0, i, 0)),
            pl.BlockSpec((H, H // 2), lambda i: (0, 0)),
            pl.BlockSpec((1, H // 2), lambda i: (0, 0)),
            pl.BlockSpec((H // 2, H), lambda i: (0, 0)),
            pl.BlockSpec((1, H), lambda i: (0, 0)),
            pl.BlockSpec((1, H), lambda i: (0, 0)),
            pl.BlockSpec((1, H), lambda i: (0, 0)),
        ],
        out_specs=pl.BlockSpec((block_rows, H), lambda i: (i, 0)),
        out_shape=jax.ShapeDtypeStruct((N, H), jnp.float32),
    )(h, agg2, w1, b1.reshape(1, H // 2), w2, b2.reshape(1, H),
      g.reshape(1, H), be.reshape(1, H))


def kernel(node_feats, edge_feats, edge_index, W_node, b_node, W_edge, b_edge,
           W1, b1, W2, b2, gamma, beta):
    pad = EP - E
    pad_src = DUMP + jnp.arange(pad, dtype=jnp.int32) % (NP - N)
    src = jnp.concatenate(
        [edge_index[0].astype(jnp.int32), pad_src]).reshape(TOTCH, CH)
    dst = jnp.concatenate(
        [edge_index[1].astype(jnp.int32),
         jnp.zeros((pad,), jnp.int32)]).reshape(TOTCH, CH)
    ef = jnp.concatenate(
        [edge_feats, jnp.zeros((pad, D_EDGE), jnp.float32)])

    h = _proj(node_feats, W_node, b_node, block_rows=1000)
    e = _proj(ef, W_edge, b_edge, block_rows=1920)

    def layer(h, wts):
        w1, bb1, w2, bb2, g, be = wts
        agg2 = _sc_agg()(h, e, src, dst)
        h = _mlp(h, agg2, w1, bb1, w2, bb2, g, be)
        return h, None

    h, _ = lax.scan(layer, h, (W1, b1, W2, b2, gamma, beta))
    return h
